# speedup vs baseline: 9.7478x; 9.7478x over previous
import functools

import jax
import jax.numpy as jnp
from jax import lax
from jax.experimental import pallas as pl
from jax.experimental.pallas import tpu as pltpu
from jax.experimental.pallas import tpu_sc as plsc

_VOCAB = 20000
_D = 768
_BM = 800


def _mlp_body(x_ref, w1_ref, b1_ref, w2_ref, b2_ref, y_ref):
    x = x_ref[...]
    h = jnp.maximum(
        jnp.dot(x, w1_ref[...], preferred_element_type=jnp.float32) + b1_ref[...],
        0.0,
    )
    y_ref[...] = (
        jnp.dot(h, w2_ref[...], preferred_element_type=jnp.float32) + b2_ref[...]
    )


def _vocab_mlp(table, W1, b1, W2, b2):
    return pl.pallas_call(
        _mlp_body,
        grid=(_VOCAB // _BM,),
        in_specs=[
            pl.BlockSpec((_BM, _D), lambda i: (i, 0)),
            pl.BlockSpec((_D, _D), lambda i: (0, 0)),
            pl.BlockSpec((1, _D), lambda i: (0, 0)),
            pl.BlockSpec((_D, _D), lambda i: (0, 0)),
            pl.BlockSpec((1, _D), lambda i: (0, 0)),
        ],
        out_specs=pl.BlockSpec((_BM, _D), lambda i: (i, 0)),
        out_shape=jax.ShapeDtypeStruct((_VOCAB, _D), jnp.float32),
    )(table, W1, b1.reshape(1, _D), W2, b2.reshape(1, _D))


def _make_gather(n_tok):
    info = plsc.get_sparse_core_info()
    nc, ns = info.num_cores, info.num_subcores
    nw = nc * ns
    assert n_tok % nw == 0
    b_per_w = n_tok // nw
    chunk = 128
    assert b_per_w % chunk == 0
    n_chunks = b_per_w // chunk
    mesh = plsc.VectorSubcoreMesh(core_axis_name="c", subcore_axis_name="s")

    @functools.partial(
        pl.kernel,
        mesh=mesh,
        out_type=jax.ShapeDtypeStruct((n_tok, _D), jnp.float32),
        scratch_types=[
            pltpu.VMEM((chunk,), jnp.int32),
            pltpu.VMEM((chunk, _D), jnp.float32),
            pltpu.SemaphoreType.DMA,
        ],
    )
    def gather_k(y_hbm, idx_hbm, out_hbm, idx_v, rows_v, sem):
        wid = lax.axis_index("s") * nc + lax.axis_index("c")
        base = wid * b_per_w

        def body(i, carry):
            off = base + i * chunk
            pltpu.sync_copy(idx_hbm.at[pl.ds(off, chunk)], idx_v)
            pltpu.async_copy(y_hbm.at[idx_v], rows_v, sem).wait()
            pltpu.sync_copy(rows_v, out_hbm.at[pl.ds(off, chunk)])
            return carry

        lax.fori_loop(0, n_chunks, body, 0)

    return gather_k


def kernel(input_ids, table, W1, b1, W2, b2):
    bsz, seq = input_ids.shape
    y = _vocab_mlp(table, W1, b1, W2, b2)
    ids_flat = input_ids.reshape(-1).astype(jnp.int32)
    gather_k = _make_gather(bsz * seq)
    out_flat = gather_k(y, ids_flat)
    return out_flat.reshape(bsz, seq, _D)


# speedup vs baseline: 10.0904x; 1.0351x over previous
import functools

import jax
import jax.numpy as jnp
from jax import lax
from jax.experimental import pallas as pl
from jax.experimental.pallas import tpu as pltpu
from jax.experimental.pallas import tpu_sc as plsc

_VOCAB = 20000
_D = 768
_BM = 800


def _mlp_body(x_ref, w1_ref, b1_ref, w2_ref, b2_ref, y_ref):
    x = x_ref[...]
    h = jnp.maximum(
        jnp.dot(x, w1_ref[...], preferred_element_type=jnp.float32) + b1_ref[...],
        0.0,
    )
    y_ref[...] = (
        jnp.dot(h, w2_ref[...], preferred_element_type=jnp.float32) + b2_ref[...]
    )


def _vocab_mlp(table, W1, b1, W2, b2):
    return pl.pallas_call(
        _mlp_body,
        grid=(_VOCAB // _BM,),
        in_specs=[
            pl.BlockSpec((_BM, _D), lambda i: (i, 0)),
            pl.BlockSpec((_D, _D), lambda i: (0, 0)),
            pl.BlockSpec((1, _D), lambda i: (0, 0)),
            pl.BlockSpec((_D, _D), lambda i: (0, 0)),
            pl.BlockSpec((1, _D), lambda i: (0, 0)),
        ],
        out_specs=pl.BlockSpec((_BM, _D), lambda i: (i, 0)),
        out_shape=jax.ShapeDtypeStruct((_VOCAB, _D), jnp.float32),
    )(table, W1, b1.reshape(1, _D), W2, b2.reshape(1, _D))


def _make_gather(n_tok):
    info = plsc.get_sparse_core_info()
    nc, ns = info.num_cores, info.num_subcores
    nw = nc * ns
    assert n_tok % nw == 0
    b_per_w = n_tok // nw
    chunk = 64
    assert b_per_w % (2 * chunk) == 0
    n_chunks = b_per_w // chunk
    n_pairs = n_chunks // 2
    mesh = plsc.VectorSubcoreMesh(core_axis_name="c", subcore_axis_name="s")

    @functools.partial(
        pl.kernel,
        mesh=mesh,
        out_type=jax.ShapeDtypeStruct((n_tok, _D), jnp.float32),
        scratch_types=[
            pltpu.VMEM((n_chunks, chunk), jnp.int32),
            pltpu.VMEM((chunk, _D), jnp.float32),
            pltpu.VMEM((chunk, _D), jnp.float32),
            pltpu.SemaphoreType.DMA,
            pltpu.SemaphoreType.DMA,
            pltpu.SemaphoreType.DMA,
            pltpu.SemaphoreType.DMA,
        ],
    )
    def gather_k(y_hbm, idx_hbm, out_hbm, idx_all, rows0, rows1, gs0, gs1, ss0, ss1):
        wid = lax.axis_index("s") * nc + lax.axis_index("c")
        base = wid * b_per_w
        pltpu.sync_copy(idx_hbm.at[wid], idx_all)
        pltpu.async_copy(y_hbm.at[idx_all.at[0]], rows0, gs0)

        def out_at(c):
            return out_hbm.at[pl.ds(base + c * chunk, chunk)]

        def body(g, carry):
            c0 = 2 * g
            c1 = c0 + 1

            @pl.when(g > 0)
            def _():
                pltpu.make_async_copy(rows1, out_at(c1), ss1).wait()

            pltpu.async_copy(y_hbm.at[idx_all.at[c1]], rows1, gs1)
            pltpu.make_async_copy(y_hbm.at[idx_all.at[c0]], rows0, gs0).wait()
            pltpu.async_copy(rows0, out_at(c0), ss0)

            @pl.when(g < n_pairs - 1)
            def _():
                pltpu.make_async_copy(rows0, out_at(c0), ss0).wait()
                pltpu.async_copy(y_hbm.at[idx_all.at[c0 + 2]], rows0, gs0)

            pltpu.make_async_copy(y_hbm.at[idx_all.at[c1]], rows1, gs1).wait()
            pltpu.async_copy(rows1, out_at(c1), ss1)
            return carry

        lax.fori_loop(0, n_pairs, body, 0)
        pltpu.make_async_copy(rows0, out_at(n_chunks - 2), ss0).wait()
        pltpu.make_async_copy(rows1, out_at(n_chunks - 1), ss1).wait()

    return gather_k, n_chunks, chunk


def kernel(input_ids, table, W1, b1, W2, b2):
    bsz, seq = input_ids.shape
    y = _vocab_mlp(table, W1, b1, W2, b2)
    gather_k, n_chunks, chunk = _make_gather(bsz * seq)
    ids = input_ids.reshape(-1, n_chunks, chunk).astype(jnp.int32)
    out_flat = gather_k(y, ids)
    return out_flat.reshape(bsz, seq, _D)
